# baseline (speedup 1.0000x reference)
import functools

import jax
import jax.numpy as jnp
from jax import lax
from jax.experimental import pallas as pl
from jax.experimental.pallas import tpu as pltpu
from jax.experimental.pallas import tpu_sc as plsc

def _take16(x, perm):
    dnums = lax.GatherDimensionNumbers(
        offset_dims=(), collapsed_slice_dims=(0,), start_index_map=(0,))
    return lax.gather(x, perm[:, None], dnums, (1,),
                      mode=lax.GatherScatterMode.PROMISE_IN_BOUNDS)


NUM_CORES = 2
NUM_SUBCORES = 16
LANES = 16
NUM_WORKERS = NUM_CORES * NUM_SUBCORES

BATCH = 16384
EMBED_DIM = 64
B_PER_W = BATCH // NUM_WORKERS
GATHER_CHUNK = 128
N_CHUNKS = B_PER_W // GATHER_CHUNK
N_GROUPS = B_PER_W // LANES


def _body(h_hbm, r_hbm, t_hbm, entity_hbm, relation_hbm, out_hbm,
          h_idx, r_idx, t_idx, h_rows, r_rows, t_rows, out_v, sem):
    wid = lax.axis_index("s") * NUM_CORES + lax.axis_index("c")
    base = wid * B_PER_W

    pltpu.sync_copy(h_hbm.at[pl.ds(base, B_PER_W)], h_idx)
    pltpu.sync_copy(r_hbm.at[pl.ds(base, B_PER_W)], r_idx)
    pltpu.sync_copy(t_hbm.at[pl.ds(base, B_PER_W)], t_idx)

    copies = []
    for k in range(N_CHUNKS):
        sl = pl.ds(k * GATHER_CHUNK, GATHER_CHUNK)
        copies.append(pltpu.async_copy(
            entity_hbm.at[h_idx.at[sl]], h_rows.at[sl], sem))
        copies.append(pltpu.async_copy(
            entity_hbm.at[t_idx.at[sl]], t_rows.at[sl], sem))
        copies.append(pltpu.async_copy(
            relation_hbm.at[r_idx.at[sl]], r_rows.at[sl], sem))
    for cp in copies:
        cp.wait()

    def group_body(g, carry):
        lane = lax.iota(jnp.int32, LANES)
        vs = []
        for j in range(LANES):
            row = g * LANES + j
            acc = None
            for k in range(EMBED_DIM // LANES):
                hv = h_rows[row, pl.ds(k * LANES, LANES)]
                rv = r_rows[row, pl.ds(k * LANES, LANES)]
                tv = t_rows[row, pl.ds(k * LANES, LANES)]
                term = hv * rv - tv
                acc = term if acc is None else acc + term
            vs.append(acc)
        for step in (1, 2, 4, 8):
            bit = (lane & step) != 0
            perm = lane ^ step
            nxt = []
            for a, b in zip(vs[0::2], vs[1::2]):
                lo = jnp.where(bit, b, a)
                hi = jnp.where(bit, a, b)
                nxt.append(lo + _take16(hi, perm))
            vs = nxt
        out_v[pl.ds(g * LANES, LANES)] = vs[0]
        return carry

    lax.fori_loop(0, N_GROUPS, group_body, 0)
    pltpu.sync_copy(out_v, out_hbm.at[pl.ds(base, B_PER_W)])


def kernel(h, r, t, entity_emb, relation_emb):
    mesh = plsc.VectorSubcoreMesh(core_axis_name="c", subcore_axis_name="s")
    run = functools.partial(
        pl.kernel,
        mesh=mesh,
        compiler_params=pltpu.CompilerParams(use_tc_tiling_on_sc=False),
        out_type=jax.ShapeDtypeStruct((BATCH,), jnp.float32),
        scratch_types=[
            pltpu.VMEM((B_PER_W,), jnp.int32),
            pltpu.VMEM((B_PER_W,), jnp.int32),
            pltpu.VMEM((B_PER_W,), jnp.int32),
            pltpu.VMEM((B_PER_W, EMBED_DIM), jnp.float32),
            pltpu.VMEM((B_PER_W, EMBED_DIM), jnp.float32),
            pltpu.VMEM((B_PER_W, EMBED_DIM), jnp.float32),
            pltpu.VMEM((B_PER_W,), jnp.float32),
            pltpu.SemaphoreType.DMA,
        ],
    )(_body)
    return run(h, r, t, entity_emb, relation_emb)


# speedup vs baseline: 1.6725x; 1.6725x over previous
import functools

import jax
import jax.numpy as jnp
from jax import lax
from jax.experimental import pallas as pl
from jax.experimental.pallas import tpu as pltpu
from jax.experimental.pallas import tpu_sc as plsc


def _take16(x, perm):
    dnums = lax.GatherDimensionNumbers(
        offset_dims=(), collapsed_slice_dims=(0,), start_index_map=(0,))
    return lax.gather(x, perm[:, None], dnums, (1,),
                      mode=lax.GatherScatterMode.PROMISE_IN_BOUNDS)


NUM_CORES = 2
NUM_SUBCORES = 16
LANES = 16
NUM_WORKERS = NUM_CORES * NUM_SUBCORES

BATCH = 16384
EMBED_DIM = 64
B_PER_W = BATCH // NUM_WORKERS
WAVE = 256
N_WAVES = B_PER_W // WAVE
GROUPS_PER_WAVE = WAVE // LANES


def _body(h_hbm, r_hbm, t_hbm, entity_hbm, relation_hbm, out_hbm,
          h_idx, r_idx, t_idx, h_rows, r_rows, t_rows, out_v, sem):
    wid = lax.axis_index("s") * NUM_CORES + lax.axis_index("c")
    base = wid * B_PER_W

    pltpu.sync_copy(h_hbm.at[pl.ds(base, B_PER_W)], h_idx)
    pltpu.sync_copy(r_hbm.at[pl.ds(base, B_PER_W)], r_idx)
    pltpu.sync_copy(t_hbm.at[pl.ds(base, B_PER_W)], t_idx)

    for w in range(N_WAVES):
        wbase = w * WAVE

        def dma_body(g, c):
            hvec = h_idx[pl.ds(wbase + g * LANES, LANES)]
            tvec = t_idx[pl.ds(wbase + g * LANES, LANES)]
            rvec = r_idx[pl.ds(wbase + g * LANES, LANES)]
            for j in range(LANES):
                row = g * LANES + j
                pltpu.async_copy(entity_hbm.at[pl.ds(hvec[j], 1)],
                                 h_rows.at[pl.ds(row, 1)], sem)
                pltpu.async_copy(entity_hbm.at[pl.ds(tvec[j], 1)],
                                 t_rows.at[pl.ds(row, 1)], sem)
                pltpu.async_copy(relation_hbm.at[pl.ds(rvec[j], 1)],
                                 r_rows.at[pl.ds(row, 1)], sem)
            return c

        lax.fori_loop(0, WAVE // LANES, dma_body, 0)
        pltpu.make_async_copy(entity_hbm.at[pl.ds(0, WAVE)], h_rows, sem).wait()
        pltpu.make_async_copy(entity_hbm.at[pl.ds(0, WAVE)], t_rows, sem).wait()
        pltpu.make_async_copy(relation_hbm.at[pl.ds(0, WAVE)], r_rows, sem).wait()

        def group_body(g, carry):
            lane = lax.iota(jnp.int32, LANES)
            vs = []
            for j in range(LANES):
                row = g * LANES + j
                acc = None
                for k in range(EMBED_DIM // LANES):
                    hv = h_rows[row, pl.ds(k * LANES, LANES)]
                    rv = r_rows[row, pl.ds(k * LANES, LANES)]
                    tv = t_rows[row, pl.ds(k * LANES, LANES)]
                    term = hv * rv - tv
                    acc = term if acc is None else acc + term
                vs.append(acc)
            for step in (1, 2, 4, 8):
                bit = (lane & step) != 0
                perm = lane ^ step
                nxt = []
                for a, b in zip(vs[0::2], vs[1::2]):
                    lo = jnp.where(bit, b, a)
                    hi = jnp.where(bit, a, b)
                    nxt.append(lo + _take16(hi, perm))
                vs = nxt
            out_v[pl.ds(wbase + g * LANES, LANES)] = vs[0]
            return carry

        lax.fori_loop(0, GROUPS_PER_WAVE, group_body, 0)

    pltpu.sync_copy(out_v, out_hbm.at[pl.ds(base, B_PER_W)])


def kernel(h, r, t, entity_emb, relation_emb):
    mesh = plsc.VectorSubcoreMesh(core_axis_name="c", subcore_axis_name="s")
    run = functools.partial(
        pl.kernel,
        mesh=mesh,
        compiler_params=pltpu.CompilerParams(use_tc_tiling_on_sc=True),
        out_type=jax.ShapeDtypeStruct((BATCH,), jnp.float32),
        scratch_types=[
            pltpu.VMEM((B_PER_W,), jnp.int32),
            pltpu.VMEM((B_PER_W,), jnp.int32),
            pltpu.VMEM((B_PER_W,), jnp.int32),
            pltpu.VMEM((WAVE, EMBED_DIM), jnp.float32),
            pltpu.VMEM((WAVE, EMBED_DIM), jnp.float32),
            pltpu.VMEM((WAVE, EMBED_DIM), jnp.float32),
            pltpu.VMEM((B_PER_W,), jnp.float32),
            pltpu.SemaphoreType.DMA,
        ],
    )(_body)
    return run(h, r, t, entity_emb, relation_emb)


# speedup vs baseline: 1.6780x; 1.0033x over previous
import functools

import jax
import jax.numpy as jnp
from jax import lax
from jax.experimental import pallas as pl
from jax.experimental.pallas import tpu as pltpu
from jax.experimental.pallas import tpu_sc as plsc


def _take16(x, perm):
    dnums = lax.GatherDimensionNumbers(
        offset_dims=(), collapsed_slice_dims=(0,), start_index_map=(0,))
    return lax.gather(x, perm[:, None], dnums, (1,),
                      mode=lax.GatherScatterMode.PROMISE_IN_BOUNDS)


NUM_CORES = 2
NUM_SUBCORES = 16
LANES = 16
NUM_WORKERS = NUM_CORES * NUM_SUBCORES

BATCH = 16384
EMBED_DIM = 64
B_PER_W = BATCH // NUM_WORKERS
WAVE = 256
N_WAVES = B_PER_W // WAVE
GROUPS_PER_WAVE = WAVE // LANES


def _body(h_hbm, r_hbm, t_hbm, entity_hbm, relation_hbm, out_hbm,
          h_idx, r_idx, t_idx, h_rows, r_rows, t_rows, out_v, sem, sem2):
    wid = lax.axis_index("s") * NUM_CORES + lax.axis_index("c")
    base = wid * B_PER_W

    pltpu.sync_copy(h_hbm.at[pl.ds(base, B_PER_W)], h_idx)
    pltpu.sync_copy(r_hbm.at[pl.ds(base, B_PER_W)], r_idx)
    pltpu.sync_copy(t_hbm.at[pl.ds(base, B_PER_W)], t_idx)

    for w in range(N_WAVES):
        wbase = w * WAVE

        def dma_body(g, c):
            hvec = h_idx[pl.ds(wbase + g * LANES, LANES)]
            tvec = t_idx[pl.ds(wbase + g * LANES, LANES)]
            rvec = r_idx[pl.ds(wbase + g * LANES, LANES)]
            for j in range(LANES):
                row = g * LANES + j
                s_a = sem if j % 2 == 0 else sem2
                s_b = sem2 if j % 2 == 0 else sem
                pltpu.async_copy(entity_hbm.at[pl.ds(hvec[j], 1)],
                                 h_rows.at[pl.ds(row, 1)], s_a)
                pltpu.async_copy(entity_hbm.at[pl.ds(tvec[j], 1)],
                                 t_rows.at[pl.ds(row, 1)], s_b)
                pltpu.async_copy(relation_hbm.at[pl.ds(rvec[j], 1)],
                                 r_rows.at[pl.ds(row, 1)], s_a)
            return c

        lax.fori_loop(0, WAVE // LANES, dma_body, 0)
        half = WAVE // 2
        for s in (sem, sem2):
            pltpu.make_async_copy(
                entity_hbm.at[pl.ds(0, half)], h_rows.at[pl.ds(0, half)], s).wait()
            pltpu.make_async_copy(
                entity_hbm.at[pl.ds(0, half)], t_rows.at[pl.ds(0, half)], s).wait()
            pltpu.make_async_copy(
                relation_hbm.at[pl.ds(0, half)], r_rows.at[pl.ds(0, half)], s).wait()

        def group_body(g, carry):
            lane = lax.iota(jnp.int32, LANES)
            vs = []
            for j in range(LANES):
                row = g * LANES + j
                acc = None
                for k in range(EMBED_DIM // LANES):
                    hv = h_rows[row, pl.ds(k * LANES, LANES)]
                    rv = r_rows[row, pl.ds(k * LANES, LANES)]
                    tv = t_rows[row, pl.ds(k * LANES, LANES)]
                    term = hv * rv - tv
                    acc = term if acc is None else acc + term
                vs.append(acc)
            for step in (1, 2, 4, 8):
                bit = (lane & step) != 0
                perm = lane ^ step
                nxt = []
                for a, b in zip(vs[0::2], vs[1::2]):
                    lo = jnp.where(bit, b, a)
                    hi = jnp.where(bit, a, b)
                    nxt.append(lo + _take16(hi, perm))
                vs = nxt
            out_v[pl.ds(wbase + g * LANES, LANES)] = vs[0]
            return carry

        lax.fori_loop(0, GROUPS_PER_WAVE, group_body, 0)

    pltpu.sync_copy(out_v, out_hbm.at[pl.ds(base, B_PER_W)])


def kernel(h, r, t, entity_emb, relation_emb):
    mesh = plsc.VectorSubcoreMesh(core_axis_name="c", subcore_axis_name="s")
    run = functools.partial(
        pl.kernel,
        mesh=mesh,
        compiler_params=pltpu.CompilerParams(use_tc_tiling_on_sc=True),
        out_type=jax.ShapeDtypeStruct((BATCH,), jnp.float32),
        scratch_types=[
            pltpu.VMEM((B_PER_W,), jnp.int32),
            pltpu.VMEM((B_PER_W,), jnp.int32),
            pltpu.VMEM((B_PER_W,), jnp.int32),
            pltpu.VMEM((WAVE, EMBED_DIM), jnp.float32),
            pltpu.VMEM((WAVE, EMBED_DIM), jnp.float32),
            pltpu.VMEM((WAVE, EMBED_DIM), jnp.float32),
            pltpu.VMEM((B_PER_W,), jnp.float32),
            pltpu.SemaphoreType.DMA,
            pltpu.SemaphoreType.DMA,
        ],
    )(_body)
    return run(h, r, t, entity_emb, relation_emb)
